# Initial kernel scaffold; baseline (speedup 1.0000x reference)
import jax
import jax.numpy as jnp
from jax.experimental import pallas as pl


def kernel(x, codebook):
    raise NotImplementedError("write your pallas kernel here")


# speedup vs baseline: 9.4794x; 9.4794x over previous
import functools

import jax
import jax.numpy as jnp
from jax import lax
from jax.experimental import pallas as pl
from jax.experimental.pallas import tpu as pltpu
from jax.experimental.pallas import tpu_sc as plsc

_K = 8192
_D = 32
_N = 8192
_TOK_BLK = 256

_NC = 2
_NS = 16
_NW = _NC * _NS
_ROWS_PER_W = _N // (_NW * 128)


def _first_index_min(d, iota):
    m = jnp.min(d, axis=1, keepdims=True)
    i = jnp.min(jnp.where(d == m, iota, jnp.int32(2 ** 30)), axis=1)
    return m[:, 0], i


def _argmin_body(xn_ref, cn_ref, x_ref, cb_ref, idx_ref):
    x = x_ref[...]
    cb = cb_ref[...]
    mm = lax.dot_general(x, cb, (((1,), (1,)), ((), ())),
                         preferred_element_type=jnp.float32)
    dist = xn_ref[...] + cn_ref[...] - 2.0 * mm
    dist = jnp.where(dist < 0, 0.0, dist)
    half = _K // 2
    iota = lax.broadcasted_iota(jnp.int32, (_TOK_BLK, half), 1)
    m0, i0 = _first_index_min(dist[:, :half], iota)
    m1, i1 = _first_index_min(dist[:, half:], iota)
    b0 = m0.astype(jnp.bfloat16).astype(jnp.float32)
    idx = jnp.where(b0 <= m1, i0, i1 + half)
    idx_ref[...] = idx.astype(jnp.int32)


def _argmin_call(xf, codebook, xn, cn):
    grid = _N // _TOK_BLK
    return pl.pallas_call(
        _argmin_body,
        grid=(grid,),
        in_specs=[
            pl.BlockSpec((_TOK_BLK, 1), lambda i: (i, 0)),
            pl.BlockSpec((1, _K), lambda i: (0, 0)),
            pl.BlockSpec((_TOK_BLK, _D), lambda i: (i, 0)),
            pl.BlockSpec((_K, _D), lambda i: (0, 0)),
        ],
        out_specs=pl.BlockSpec((_TOK_BLK,), lambda i: (i,)),
        out_shape=jax.ShapeDtypeStruct((_N,), jnp.int32),
    )(xn, cn, xf, codebook)


def _sc_gather_counts(codebook, idx2d, zeros_k):
    mesh = plsc.VectorSubcoreMesh(core_axis_name="c", subcore_axis_name="s")

    @functools.partial(
        pl.kernel,
        out_type=(
            jax.ShapeDtypeStruct((_N, _D), jnp.float32),
            jax.ShapeDtypeStruct((_NC, _K), jnp.float32),
        ),
        mesh=mesh,
        compiler_params=pltpu.CompilerParams(use_tc_tiling_on_sc=False),
        scratch_types=[
            pltpu.VMEM((_ROWS_PER_W, 128), jnp.int32),
            pltpu.VMEM((128, _D), jnp.float32),
            pltpu.VMEM((128,), jnp.float32),
            pltpu.VMEM_SHARED((_K,), jnp.float32),
            pltpu.SemaphoreType.DMA,
        ],
    )
    def k(cb_hbm, idx_hbm, zk_hbm, z_hbm, cnt_hbm, idx_v, rows_v, ones_v,
          cnt_sh, sem):
        cid = lax.axis_index("c")
        sid = lax.axis_index("s")
        wid = sid * _NC + cid
        pltpu.sync_copy(idx_hbm.at[pl.ds(wid * _ROWS_PER_W, _ROWS_PER_W)],
                        idx_v)
        for i in range(128 // 16):
            ones_v[pl.ds(i * 16, 16)] = jnp.full((16,), 1.0, jnp.float32)

        @pl.when(sid == 0)
        def _():
            pltpu.sync_copy(zk_hbm, cnt_sh)

        plsc.subcore_barrier()
        for j in range(_ROWS_PER_W):
            pltpu.async_copy(cb_hbm.at[idx_v.at[j]], rows_v, sem).wait()
            pltpu.sync_copy(
                rows_v, z_hbm.at[pl.ds(wid * _ROWS_PER_W * 128 + j * 128, 128)])
            pltpu.sync_copy(ones_v, cnt_sh.at[idx_v.at[j]], add=True)
        plsc.subcore_barrier()

        @pl.when(sid == 0)
        def _():
            pltpu.sync_copy(cnt_sh, cnt_hbm.at[cid])

    return k(codebook, idx2d, zeros_k)


def _loss_body(x_ref, z_ref, cnt_ref, zout_ref, loss_ref, perp_ref):
    xv = x_ref[...]
    d = z_ref[...] - xv
    zout_ref[...] = xv + d
    loss_ref[...] = jnp.reshape(jnp.sum(d * d) * (1.0 / (_N * _D)), (1, 1))
    c = cnt_ref[0:1, :] + cnt_ref[1:2, :]
    p = c * (1.0 / _N)
    ent = jnp.sum(p * jnp.log(p + 1e-10))
    perp_ref[...] = jnp.reshape(jnp.exp(-ent), (1, 1))


def _loss_call(xf, z, counts):
    return pl.pallas_call(
        _loss_body,
        out_shape=(
            jax.ShapeDtypeStruct((_N, _D), jnp.float32),
            jax.ShapeDtypeStruct((1, 1), jnp.float32),
            jax.ShapeDtypeStruct((1, 1), jnp.float32),
        ),
    )(xf, z, counts)


def kernel(x, codebook):
    xf = x.reshape(-1, _D)
    xn = jnp.sum(xf ** 2, axis=1).reshape(-1, 1)
    cn = jnp.sum(codebook ** 2, axis=1).reshape(1, -1)
    idx = _argmin_call(xf, codebook, xn, cn)
    idx2d = idx.reshape(_N // 128, 128)
    zeros_k = jnp.zeros((_K,), jnp.float32)
    z_flat, counts = _sc_gather_counts(codebook, idx2d, zeros_k)
    zout, loss, perp = _loss_call(xf, z_flat, counts)
    z = zout.reshape(x.shape)
    loss = loss.reshape(())
    perp = perp.reshape(())
    return (z, loss, loss, perp)


# speedup vs baseline: 10.1882x; 1.0748x over previous
import functools

import jax
import jax.numpy as jnp
from jax import lax
from jax.experimental import pallas as pl
from jax.experimental.pallas import tpu as pltpu
from jax.experimental.pallas import tpu_sc as plsc

_K = 8192
_D = 32
_N = 8192
_TOK_BLK = 256

_NC = 2
_NS = 16
_NW = _NC * _NS
_ROWS_PER_W = _N // (_NW * 128)


def _first_index_min(d, iota_f):
    m = jnp.min(d, axis=1, keepdims=True)
    i = jnp.min(jnp.where(d == m, iota_f, jnp.float32(1e9)), axis=1)
    return m[:, 0], i


def _argmin_body(xn_ref, cn_ref, xm2_ref, cb_ref, idx_ref):
    xm2 = xm2_ref[...]
    cb = cb_ref[...]
    mm2 = lax.dot_general(xm2, cb, (((1,), (1,)), ((), ())),
                          preferred_element_type=jnp.float32)
    dist = xn_ref[...] + cn_ref[...] + mm2
    dist = jnp.maximum(dist, 0.0)
    half = _K // 2
    iota_f = lax.broadcasted_iota(
        jnp.int32, (_TOK_BLK, half), 1).astype(jnp.float32)
    m0, i0 = _first_index_min(dist[:, :half], iota_f)
    m1, i1 = _first_index_min(dist[:, half:], iota_f)
    b0 = m0.astype(jnp.bfloat16).astype(jnp.float32)
    idx = jnp.where(b0 <= m1, i0, i1 + jnp.float32(half))
    idx_ref[...] = idx.astype(jnp.int32)


def _argmin_call(xf, codebook, xn, cn):
    grid = _N // _TOK_BLK
    return pl.pallas_call(
        _argmin_body,
        grid=(grid,),
        in_specs=[
            pl.BlockSpec((_TOK_BLK, 1), lambda i: (i, 0)),
            pl.BlockSpec((1, _K), lambda i: (0, 0)),
            pl.BlockSpec((_TOK_BLK, _D), lambda i: (i, 0)),
            pl.BlockSpec((_K, _D), lambda i: (0, 0)),
        ],
        out_specs=pl.BlockSpec((_TOK_BLK,), lambda i: (i,)),
        out_shape=jax.ShapeDtypeStruct((_N,), jnp.int32),
    )(xn, cn, xf, codebook)


def _sc_gather_counts(codebook, idx2d, zeros_k):
    mesh = plsc.VectorSubcoreMesh(core_axis_name="c", subcore_axis_name="s")

    @functools.partial(
        pl.kernel,
        out_type=(
            jax.ShapeDtypeStruct((_N, _D), jnp.float32),
            jax.ShapeDtypeStruct((_NC, _K), jnp.float32),
        ),
        mesh=mesh,
        compiler_params=pltpu.CompilerParams(use_tc_tiling_on_sc=False),
        scratch_types=[
            pltpu.VMEM((_ROWS_PER_W, 128), jnp.int32),
            pltpu.VMEM((128, _D), jnp.float32),
            pltpu.VMEM((128,), jnp.float32),
            pltpu.VMEM_SHARED((_K,), jnp.float32),
            pltpu.SemaphoreType.DMA,
        ],
    )
    def k(cb_hbm, idx_hbm, zk_hbm, z_hbm, cnt_hbm, idx_v, rows_v, ones_v,
          cnt_sh, sem):
        cid = lax.axis_index("c")
        sid = lax.axis_index("s")
        wid = sid * _NC + cid
        pltpu.sync_copy(idx_hbm.at[pl.ds(wid * _ROWS_PER_W, _ROWS_PER_W)],
                        idx_v)
        for i in range(128 // 16):
            ones_v[pl.ds(i * 16, 16)] = jnp.full((16,), 1.0, jnp.float32)

        @pl.when(sid == 0)
        def _():
            pltpu.sync_copy(zk_hbm, cnt_sh)

        plsc.subcore_barrier()
        for j in range(_ROWS_PER_W):
            pltpu.async_copy(cb_hbm.at[idx_v.at[j]], rows_v, sem).wait()
            pltpu.sync_copy(
                rows_v, z_hbm.at[pl.ds(wid * _ROWS_PER_W * 128 + j * 128, 128)])
            pltpu.sync_copy(ones_v, cnt_sh.at[idx_v.at[j]], add=True)
        plsc.subcore_barrier()

        @pl.when(sid == 0)
        def _():
            pltpu.sync_copy(cnt_sh, cnt_hbm.at[cid])

    return k(codebook, idx2d, zeros_k)


def _loss_body(x_ref, z_ref, cnt_ref, zout_ref, loss_ref, perp_ref):
    xv = x_ref[...]
    d = z_ref[...] - xv
    zout_ref[...] = xv + d
    loss_ref[...] = jnp.reshape(jnp.sum(d * d) * (1.0 / (_N * _D)), (1, 1))
    c = cnt_ref[0:1, :] + cnt_ref[1:2, :]
    p = c * (1.0 / _N)
    ent = jnp.sum(p * jnp.log(p + 1e-10))
    perp_ref[...] = jnp.reshape(jnp.exp(-ent), (1, 1))


def _loss_call(xf, z, counts):
    return pl.pallas_call(
        _loss_body,
        out_shape=(
            jax.ShapeDtypeStruct((_N, _D), jnp.float32),
            jax.ShapeDtypeStruct((1, 1), jnp.float32),
            jax.ShapeDtypeStruct((1, 1), jnp.float32),
        ),
    )(xf, z, counts)


def kernel(x, codebook):
    xf = x.reshape(-1, _D)
    xn = jnp.sum(xf ** 2, axis=1).reshape(-1, 1)
    cn = jnp.sum(codebook ** 2, axis=1).reshape(1, -1)
    xm2 = xf * jnp.float32(-2.0)
    idx = _argmin_call(xm2, codebook, xn, cn)
    idx2d = idx.reshape(_N // 128, 128)
    zeros_k = jnp.zeros((_K,), jnp.float32)
    z_flat, counts = _sc_gather_counts(codebook, idx2d, zeros_k)
    zout, loss, perp = _loss_call(xf, z_flat, counts)
    z = zout.reshape(x.shape)
    loss = loss.reshape(())
    perp = perp.reshape(())
    return (z, loss, loss, perp)


# speedup vs baseline: 10.9037x; 1.0702x over previous
import functools

import jax
import jax.numpy as jnp
from jax import lax
from jax.experimental import pallas as pl
from jax.experimental.pallas import tpu as pltpu
from jax.experimental.pallas import tpu_sc as plsc

_K = 8192
_D = 32
_N = 8192
_TOK_BLK = 512

_NC = 2
_NS = 16
_NW = _NC * _NS
_ROWS_PER_W = _N // (_NW * 128)


def _first_index_min(d, iota_f):
    m = jnp.min(d, axis=1, keepdims=True)
    i = jnp.min(jnp.where(d == m, iota_f, jnp.float32(1e9)), axis=1)
    return m[:, 0], i


def _argmin_body(xn_ref, cn_ref, xm2_ref, cb_ref, idx_ref):
    xm2 = xm2_ref[...]
    cb = cb_ref[...]
    mm2 = lax.dot_general(xm2, cb, (((1,), (1,)), ((), ())),
                          preferred_element_type=jnp.float32)
    dist = xn_ref[...] + cn_ref[...] + mm2
    dist = jnp.maximum(dist, 0.0)
    half = _K // 2
    iota_f = lax.broadcasted_iota(
        jnp.int32, (_TOK_BLK, half), 1).astype(jnp.float32)
    m0, i0 = _first_index_min(dist[:, :half], iota_f)
    m1, i1 = _first_index_min(dist[:, half:], iota_f)
    b0 = m0.astype(jnp.bfloat16).astype(jnp.float32)
    idx = jnp.where(b0 <= m1, i0, i1 + jnp.float32(half))
    idx_ref[...] = idx.astype(jnp.int32)


def _argmin_call(xf, codebook, xn, cn):
    grid = _N // _TOK_BLK
    return pl.pallas_call(
        _argmin_body,
        grid=(grid,),
        in_specs=[
            pl.BlockSpec((_TOK_BLK, 1), lambda i: (i, 0)),
            pl.BlockSpec((1, _K), lambda i: (0, 0)),
            pl.BlockSpec((_TOK_BLK, _D), lambda i: (i, 0)),
            pl.BlockSpec((_K, _D), lambda i: (0, 0)),
        ],
        out_specs=pl.BlockSpec((_TOK_BLK,), lambda i: (i,)),
        out_shape=jax.ShapeDtypeStruct((_N,), jnp.int32),
    )(xn, cn, xf, codebook)


def _sc_gather_counts(codebook, idx2d, zeros_k):
    mesh = plsc.VectorSubcoreMesh(core_axis_name="c", subcore_axis_name="s")

    @functools.partial(
        pl.kernel,
        out_type=(
            jax.ShapeDtypeStruct((_N, _D), jnp.float32),
            jax.ShapeDtypeStruct((_NC, _K), jnp.float32),
        ),
        mesh=mesh,
        compiler_params=pltpu.CompilerParams(use_tc_tiling_on_sc=False),
        scratch_types=[
            pltpu.VMEM((_ROWS_PER_W, 128), jnp.int32),
            pltpu.VMEM((128, _D), jnp.float32),
            pltpu.VMEM((128,), jnp.float32),
            pltpu.VMEM_SHARED((_K,), jnp.float32),
            pltpu.SemaphoreType.DMA,
        ],
    )
    def k(cb_hbm, idx_hbm, zk_hbm, z_hbm, cnt_hbm, idx_v, rows_v, ones_v,
          cnt_sh, sem):
        cid = lax.axis_index("c")
        sid = lax.axis_index("s")
        wid = sid * _NC + cid
        pltpu.sync_copy(idx_hbm.at[pl.ds(wid * _ROWS_PER_W, _ROWS_PER_W)],
                        idx_v)
        for i in range(128 // 16):
            ones_v[pl.ds(i * 16, 16)] = jnp.full((16,), 1.0, jnp.float32)

        @pl.when(sid == 0)
        def _():
            pltpu.sync_copy(zk_hbm, cnt_sh)

        plsc.subcore_barrier()
        for j in range(_ROWS_PER_W):
            pltpu.async_copy(cb_hbm.at[idx_v.at[j]], rows_v, sem).wait()
            pltpu.sync_copy(
                rows_v, z_hbm.at[pl.ds(wid * _ROWS_PER_W * 128 + j * 128, 128)])
            pltpu.sync_copy(ones_v, cnt_sh.at[idx_v.at[j]], add=True)
        plsc.subcore_barrier()

        @pl.when(sid == 0)
        def _():
            pltpu.sync_copy(cnt_sh, cnt_hbm.at[cid])

    return k(codebook, idx2d, zeros_k)


def _loss_body(x_ref, z_ref, cnt_ref, zout_ref, loss_ref, perp_ref):
    xv = x_ref[...]
    d = z_ref[...] - xv
    zout_ref[...] = xv + d
    loss_ref[...] = jnp.reshape(jnp.sum(d * d) * (1.0 / (_N * _D)), (1, 1))
    c = cnt_ref[0:1, :] + cnt_ref[1:2, :]
    p = c * (1.0 / _N)
    ent = jnp.sum(p * jnp.log(p + 1e-10))
    perp_ref[...] = jnp.reshape(jnp.exp(-ent), (1, 1))


def _loss_call(xf, z, counts):
    return pl.pallas_call(
        _loss_body,
        out_shape=(
            jax.ShapeDtypeStruct((_N, _D), jnp.float32),
            jax.ShapeDtypeStruct((1, 1), jnp.float32),
            jax.ShapeDtypeStruct((1, 1), jnp.float32),
        ),
    )(xf, z, counts)


def kernel(x, codebook):
    xf = x.reshape(-1, _D)
    xn = jnp.sum(xf ** 2, axis=1).reshape(-1, 1)
    cn = jnp.sum(codebook ** 2, axis=1).reshape(1, -1)
    xm2 = xf * jnp.float32(-2.0)
    idx = _argmin_call(xm2, codebook, xn, cn)
    idx2d = idx.reshape(_N // 128, 128)
    zeros_k = jnp.zeros((_K,), jnp.float32)
    z_flat, counts = _sc_gather_counts(codebook, idx2d, zeros_k)
    zout, loss, perp = _loss_call(xf, z_flat, counts)
    z = zout.reshape(x.shape)
    loss = loss.reshape(())
    perp = perp.reshape(())
    return (z, loss, loss, perp)


# speedup vs baseline: 11.7400x; 1.0767x over previous
import functools

import jax
import jax.numpy as jnp
from jax import lax
from jax.experimental import pallas as pl
from jax.experimental.pallas import tpu as pltpu
from jax.experimental.pallas import tpu_sc as plsc

_K = 8192
_D = 32
_N = 8192
_TOK_BLK = 1024

_NC = 2
_NS = 16
_NW = _NC * _NS
_ROWS_PER_W = _N // (_NW * 128)


def _first_index_min(d, iota_f):
    m = jnp.min(d, axis=1, keepdims=True)
    i = jnp.min(jnp.where(d == m, iota_f, jnp.float32(1e9)), axis=1)
    return m[:, 0], i


def _argmin_body(xn_ref, cn_ref, x_ref, cb_ref, idx_ref):
    xm2 = x_ref[...] * jnp.float32(-2.0)
    cb = cb_ref[...]
    mm2 = lax.dot_general(xm2, cb, (((1,), (1,)), ((), ())),
                          preferred_element_type=jnp.float32)
    dist = xn_ref[...] + cn_ref[...] + mm2
    dist = jnp.maximum(dist, 0.0)
    half = _K // 2
    iota_f = lax.broadcasted_iota(
        jnp.int32, (_TOK_BLK, half), 1).astype(jnp.float32)
    m0, i0 = _first_index_min(dist[:, :half], iota_f)
    m1, i1 = _first_index_min(dist[:, half:], iota_f)
    b0 = m0.astype(jnp.bfloat16).astype(jnp.float32)
    idx = jnp.where(b0 <= m1, i0, i1 + jnp.float32(half))
    idx_ref[...] = idx.astype(jnp.int32)


def _argmin_call(xf, codebook, xn, cn):
    grid = _N // _TOK_BLK
    return pl.pallas_call(
        _argmin_body,
        grid=(grid,),
        in_specs=[
            pl.BlockSpec((_TOK_BLK, 1), lambda i: (i, 0)),
            pl.BlockSpec((1, _K), lambda i: (0, 0)),
            pl.BlockSpec((_TOK_BLK, _D), lambda i: (i, 0)),
            pl.BlockSpec((_K, _D), lambda i: (0, 0)),
        ],
        out_specs=pl.BlockSpec((_TOK_BLK,), lambda i: (i,)),
        out_shape=jax.ShapeDtypeStruct((_N,), jnp.int32),
    )(xn, cn, xf, codebook)


def _sc_gather_counts(codebook, idx2d, zeros_k):
    mesh = plsc.VectorSubcoreMesh(core_axis_name="c", subcore_axis_name="s")

    @functools.partial(
        pl.kernel,
        out_type=(
            jax.ShapeDtypeStruct((_N, _D), jnp.float32),
            jax.ShapeDtypeStruct((_NC, _K), jnp.float32),
        ),
        mesh=mesh,
        compiler_params=pltpu.CompilerParams(use_tc_tiling_on_sc=False),
        scratch_types=[
            pltpu.VMEM((_ROWS_PER_W, 128), jnp.int32),
            pltpu.VMEM((128, _D), jnp.float32),
            pltpu.VMEM((128,), jnp.float32),
            pltpu.VMEM_SHARED((_K,), jnp.float32),
            pltpu.SemaphoreType.DMA,
        ],
    )
    def k(cb_hbm, idx_hbm, zk_hbm, z_hbm, cnt_hbm, idx_v, rows_v, ones_v,
          cnt_sh, sem):
        cid = lax.axis_index("c")
        sid = lax.axis_index("s")
        wid = sid * _NC + cid
        pltpu.sync_copy(idx_hbm.at[pl.ds(wid * _ROWS_PER_W, _ROWS_PER_W)],
                        idx_v)
        for i in range(128 // 16):
            ones_v[pl.ds(i * 16, 16)] = jnp.full((16,), 1.0, jnp.float32)

        @pl.when(sid == 0)
        def _():
            pltpu.sync_copy(zk_hbm, cnt_sh)

        plsc.subcore_barrier()
        for j in range(_ROWS_PER_W):
            pltpu.async_copy(cb_hbm.at[idx_v.at[j]], rows_v, sem).wait()
            pltpu.sync_copy(
                rows_v, z_hbm.at[pl.ds(wid * _ROWS_PER_W * 128 + j * 128, 128)])
            pltpu.sync_copy(ones_v, cnt_sh.at[idx_v.at[j]], add=True)
        plsc.subcore_barrier()

        @pl.when(sid == 0)
        def _():
            pltpu.sync_copy(cnt_sh, cnt_hbm.at[cid])

    return k(codebook, idx2d, zeros_k)


def _loss_body(x_ref, z_ref, cnt_ref, zout_ref, loss_ref, perp_ref):
    xv = x_ref[...]
    d = z_ref[...] - xv
    zout_ref[...] = xv + d
    loss_ref[...] = jnp.reshape(jnp.sum(d * d) * (1.0 / (_N * _D)), (1, 1))
    c = cnt_ref[0:1, :] + cnt_ref[1:2, :]
    p = c * (1.0 / _N)
    ent = jnp.sum(p * jnp.log(p + 1e-10))
    perp_ref[...] = jnp.reshape(jnp.exp(-ent), (1, 1))


def _loss_call(xf, z, counts):
    return pl.pallas_call(
        _loss_body,
        out_shape=(
            jax.ShapeDtypeStruct((_N, _D), jnp.float32),
            jax.ShapeDtypeStruct((1, 1), jnp.float32),
            jax.ShapeDtypeStruct((1, 1), jnp.float32),
        ),
    )(xf, z, counts)


def kernel(x, codebook):
    xf = x.reshape(-1, _D)
    xn = jnp.sum(xf ** 2, axis=1).reshape(-1, 1)
    cn = jnp.sum(codebook ** 2, axis=1).reshape(1, -1)
    idx = _argmin_call(xf, codebook, xn, cn)
    idx2d = idx.reshape(_N // 128, 128)
    zeros_k = jnp.zeros((_K,), jnp.float32)
    z_flat, counts = _sc_gather_counts(codebook, idx2d, zeros_k)
    zout, loss, perp = _loss_call(xf, z_flat, counts)
    z = zout.reshape(x.shape)
    loss = loss.reshape(())
    perp = perp.reshape(())
    return (z, loss, loss, perp)


# speedup vs baseline: 11.9156x; 1.0150x over previous
import functools

import jax
import jax.numpy as jnp
from jax import lax
from jax.experimental import pallas as pl
from jax.experimental.pallas import tpu as pltpu
from jax.experimental.pallas import tpu_sc as plsc

_K = 8192
_D = 32
_N = 8192
_TOK_BLK = 1024

_NC = 2
_NS = 16
_NW = _NC * _NS
_ROWS_PER_W = _N // (_NW * 128)


def _first_index_min(d, iota_f):
    m = jnp.min(d, axis=1, keepdims=True)
    i = jnp.min(jnp.where(d == m, iota_f, jnp.float32(1e9)), axis=1)
    return m[:, 0], i


def _argmin_body(xn_ref, cn_ref, x_ref, cb_ref, idx_ref):
    xm2 = x_ref[...] * jnp.float32(-2.0)
    cb = cb_ref[...]
    mm2 = lax.dot_general(xm2, cb, (((1,), (1,)), ((), ())),
                          preferred_element_type=jnp.float32)
    dist = xn_ref[...] + cn_ref[...] + mm2
    dist = jnp.maximum(dist, 0.0)
    half = _K // 2
    iota_f = lax.broadcasted_iota(
        jnp.int32, (_TOK_BLK, half), 1).astype(jnp.float32)
    m0, i0 = _first_index_min(dist[:, :half], iota_f)
    m1, i1 = _first_index_min(dist[:, half:], iota_f)
    b0 = m0.astype(jnp.bfloat16).astype(jnp.float32)
    idx = jnp.where(b0 <= m1, i0, i1 + jnp.float32(half))
    idx_ref[...] = idx.astype(jnp.int32)


def _argmin_call(xf, codebook, xn, cn):
    grid = _N // _TOK_BLK
    return pl.pallas_call(
        _argmin_body,
        grid=(grid,),
        in_specs=[
            pl.BlockSpec((_TOK_BLK, 1), lambda i: (i, 0)),
            pl.BlockSpec((1, _K), lambda i: (0, 0)),
            pl.BlockSpec((_TOK_BLK, _D), lambda i: (i, 0)),
            pl.BlockSpec((_K, _D), lambda i: (0, 0)),
        ],
        out_specs=pl.BlockSpec((_TOK_BLK,), lambda i: (i,)),
        out_shape=jax.ShapeDtypeStruct((_N,), jnp.int32),
    )(xn, cn, xf, codebook)


def _sc_gather_counts(codebook, idx2d, zeros_k):
    mesh = plsc.VectorSubcoreMesh(core_axis_name="c", subcore_axis_name="s")

    @functools.partial(
        pl.kernel,
        out_type=(
            jax.ShapeDtypeStruct((_N, _D), jnp.float32),
            jax.ShapeDtypeStruct((_NC, _K), jnp.float32),
        ),
        mesh=mesh,
        compiler_params=pltpu.CompilerParams(use_tc_tiling_on_sc=False),
        scratch_types=[
            pltpu.VMEM((_ROWS_PER_W, 128), jnp.int32),
            pltpu.VMEM((_ROWS_PER_W, 128, _D), jnp.float32),
            pltpu.VMEM((128,), jnp.float32),
            pltpu.VMEM_SHARED((_K,), jnp.float32),
            pltpu.SemaphoreType.DMA,
            pltpu.SemaphoreType.DMA,
        ],
    )
    def k(cb_hbm, idx_hbm, zk_hbm, z_hbm, cnt_hbm, idx_v, rows_v, ones_v,
          cnt_sh, gsem, wsem):
        cid = lax.axis_index("c")
        sid = lax.axis_index("s")
        wid = sid * _NC + cid
        pltpu.sync_copy(idx_hbm.at[pl.ds(wid * _ROWS_PER_W, _ROWS_PER_W)],
                        idx_v)
        gathers = [
            pltpu.async_copy(cb_hbm.at[idx_v.at[j]], rows_v.at[j], gsem)
            for j in range(_ROWS_PER_W)
        ]
        for i in range(128 // 16):
            ones_v[pl.ds(i * 16, 16)] = jnp.full((16,), 1.0, jnp.float32)

        @pl.when(sid == 0)
        def _():
            pltpu.sync_copy(zk_hbm, cnt_sh)

        plsc.subcore_barrier()
        writes = []
        for j in range(_ROWS_PER_W):
            gathers[j].wait()
            writes.append(pltpu.async_copy(
                rows_v.at[j],
                z_hbm.at[pl.ds(wid * _ROWS_PER_W * 128 + j * 128, 128)], wsem))
            pltpu.sync_copy(ones_v, cnt_sh.at[idx_v.at[j]], add=True)
        for w in writes:
            w.wait()
        plsc.subcore_barrier()

        @pl.when(sid == 0)
        def _():
            pltpu.sync_copy(cnt_sh, cnt_hbm.at[cid])

    return k(codebook, idx2d, zeros_k)


def _loss_body(x_ref, z_ref, cnt_ref, zout_ref, loss_ref, perp_ref):
    xv = x_ref[...]
    d = z_ref[...] - xv
    zout_ref[...] = xv + d
    loss_ref[...] = jnp.reshape(jnp.sum(d * d) * (1.0 / (_N * _D)), (1, 1))
    c = cnt_ref[0:1, :] + cnt_ref[1:2, :]
    p = c * (1.0 / _N)
    ent = jnp.sum(p * jnp.log(p + 1e-10))
    perp_ref[...] = jnp.reshape(jnp.exp(-ent), (1, 1))


def _loss_call(xf, z, counts):
    return pl.pallas_call(
        _loss_body,
        out_shape=(
            jax.ShapeDtypeStruct((_N, _D), jnp.float32),
            jax.ShapeDtypeStruct((1, 1), jnp.float32),
            jax.ShapeDtypeStruct((1, 1), jnp.float32),
        ),
    )(xf, z, counts)


def kernel(x, codebook):
    xf = x.reshape(-1, _D)
    xn = jnp.sum(xf ** 2, axis=1).reshape(-1, 1)
    cn = jnp.sum(codebook ** 2, axis=1).reshape(1, -1)
    idx = _argmin_call(xf, codebook, xn, cn)
    idx2d = idx.reshape(_N // 128, 128)
    zeros_k = jnp.zeros((_K,), jnp.float32)
    z_flat, counts = _sc_gather_counts(codebook, idx2d, zeros_k)
    zout, loss, perp = _loss_call(xf, z_flat, counts)
    z = zout.reshape(x.shape)
    loss = loss.reshape(())
    perp = perp.reshape(())
    return (z, loss, loss, perp)
